# baseline (speedup 1.0000x reference)
import jax
import jax.numpy as jnp
from jax import lax
from jax.experimental import pallas as pl
from jax.experimental.pallas import tpu as pltpu

_TM = 200


def _fused_body(x_ref, adj_ref, w1_ref, b1_ref, w2_ref, b2_ref, out_ref,
                s1_ref, s2_ref):
    i = pl.program_id(0)
    p = pl.num_programs(0) // 2

    @pl.when(i == 0)
    def _():
        s1_ref[...] = jnp.dot(
            x_ref[...], w1_ref[...], preferred_element_type=jnp.float32)

    a = adj_ref[...]

    @pl.when(i < p)
    def _():
        acc = jnp.dot(a, s1_ref[...], preferred_element_type=jnp.float32)
        h = jnp.maximum(acc + b1_ref[...], 0.0)
        s2_ref[pl.ds(i * _TM, _TM), :] = jnp.dot(
            h, w2_ref[...], preferred_element_type=jnp.float32)

    @pl.when(i >= p)
    def _():
        acc = jnp.dot(a, s2_ref[...], preferred_element_type=jnp.float32)
        out_ref[...] = acc + b2_ref[...]


def kernel(x, adj, W1, b1, W2, b2):
    n, nfeat = x.shape
    nhid = W1.shape[1]
    nout = W2.shape[1]
    b1r = b1.reshape(1, nhid)
    b2r = b2.reshape(1, nout)

    p = n // _TM
    grid = (2 * p,)

    out = pl.pallas_call(
        _fused_body,
        grid=grid,
        in_specs=[
            pl.BlockSpec((n, nfeat), lambda i: (0, 0)),
            pl.BlockSpec((_TM, n), lambda i: (i % p, 0)),
            pl.BlockSpec((nfeat, nhid), lambda i: (0, 0)),
            pl.BlockSpec((1, nhid), lambda i: (0, 0)),
            pl.BlockSpec((nhid, nout), lambda i: (0, 0)),
            pl.BlockSpec((1, nout), lambda i: (0, 0)),
        ],
        out_specs=pl.BlockSpec((_TM, nout), lambda i: (lax.max(i - p, 0), 0)),
        out_shape=jax.ShapeDtypeStruct((n, nout), jnp.float32),
        scratch_shapes=[
            pltpu.VMEM((n, nhid), jnp.float32),
            pltpu.VMEM((n, nout), jnp.float32),
        ],
        compiler_params=pltpu.CompilerParams(
            dimension_semantics=("arbitrary",)),
    )(x, adj, W1, b1r, W2, b2r)

    return out


# speedup vs baseline: 1.0315x; 1.0315x over previous
import jax
import jax.numpy as jnp
from jax.experimental import pallas as pl
from jax.experimental.pallas import tpu as pltpu

_TM1 = 200
_TM2 = 400


def _pass1_body(x_ref, adj_ref, w1_ref, b1_ref, w2_ref, abf_ref, s2_ref,
                s1_ref):
    i = pl.program_id(0)

    @pl.when(i == 0)
    def _():
        s1_ref[...] = jnp.dot(
            x_ref[...].astype(jnp.bfloat16), w1_ref[...],
            preferred_element_type=jnp.float32).astype(jnp.bfloat16)

    a = adj_ref[...].astype(jnp.bfloat16)
    abf_ref[...] = a
    acc = jnp.dot(a, s1_ref[...], preferred_element_type=jnp.float32)
    h = jnp.maximum(acc + b1_ref[...], 0.0).astype(jnp.bfloat16)
    s2_ref[...] = jnp.dot(
        h, w2_ref[...], preferred_element_type=jnp.float32
    ).astype(jnp.bfloat16)


def _pass2_body(abf_ref, s2_ref, b2_ref, out_ref):
    acc = jnp.dot(abf_ref[...], s2_ref[...], preferred_element_type=jnp.float32)
    out_ref[...] = acc + b2_ref[...]


def kernel(x, adj, W1, b1, W2, b2):
    n, nfeat = x.shape
    nhid = W1.shape[1]
    nout = W2.shape[1]
    w1b = W1.astype(jnp.bfloat16)
    w2b = W2.astype(jnp.bfloat16)
    b1r = b1.reshape(1, nhid)
    b2r = b2.reshape(1, nout)

    abf, s2 = pl.pallas_call(
        _pass1_body,
        grid=(n // _TM1,),
        in_specs=[
            pl.BlockSpec((n, nfeat), lambda i: (0, 0)),
            pl.BlockSpec((_TM1, n), lambda i: (i, 0)),
            pl.BlockSpec((nfeat, nhid), lambda i: (0, 0)),
            pl.BlockSpec((1, nhid), lambda i: (0, 0)),
            pl.BlockSpec((nhid, nout), lambda i: (0, 0)),
        ],
        out_specs=[
            pl.BlockSpec((_TM1, n), lambda i: (i, 0)),
            pl.BlockSpec((_TM1, nout), lambda i: (i, 0)),
        ],
        out_shape=[
            jax.ShapeDtypeStruct((n, n), jnp.bfloat16),
            jax.ShapeDtypeStruct((n, nout), jnp.bfloat16),
        ],
        scratch_shapes=[pltpu.VMEM((n, nhid), jnp.bfloat16)],
        compiler_params=pltpu.CompilerParams(
            dimension_semantics=("arbitrary",)),
    )(x, adj, w1b, b1r, w2b)

    out = pl.pallas_call(
        _pass2_body,
        grid=(n // _TM2,),
        in_specs=[
            pl.BlockSpec((_TM2, n), lambda i: (i, 0)),
            pl.BlockSpec((n, nout), lambda i: (0, 0)),
            pl.BlockSpec((1, nout), lambda i: (0, 0)),
        ],
        out_specs=pl.BlockSpec((_TM2, nout), lambda i: (i, 0)),
        out_shape=jax.ShapeDtypeStruct((n, nout), jnp.float32),
        compiler_params=pltpu.CompilerParams(
            dimension_semantics=("arbitrary",)),
    )(abf, s2, b2r)

    return out


# speedup vs baseline: 1.1074x; 1.0736x over previous
import jax
import jax.numpy as jnp
from jax import lax
from jax.experimental import pallas as pl
from jax.experimental.pallas import tpu as pltpu

_TM = 400


def _body(x_ref, adj_ref, w1_ref, b1_ref, w2_ref, b2_ref, out_ref,
          s1_ref, s2_ref, abuf_ref, sem_ref):
    n = x_ref.shape[0]
    nt = n // _TM
    total = 2 * nt

    s1_ref[...] = jnp.dot(
        x_ref[...].astype(jnp.bfloat16), w1_ref[...],
        preferred_element_type=jnp.float32).astype(jnp.bfloat16)

    def _copy(t, slot):
        row = lax.rem(t, nt) * _TM
        return pltpu.make_async_copy(
            adj_ref.at[pl.ds(row, _TM), :], abuf_ref.at[slot],
            sem_ref.at[slot])

    _copy(0, 0).start()

    def _loop(i, carry):
        slot = lax.rem(i, 2)
        nxt = lax.rem(i + 1, 2)

        @pl.when(i + 1 < total)
        def _():
            _copy(i + 1, nxt).start()

        _copy(i, slot).wait()
        a = abuf_ref[slot].astype(jnp.bfloat16)
        row = lax.rem(i, nt) * _TM

        @pl.when(i < nt)
        def _():
            acc = jnp.dot(a, s1_ref[...], preferred_element_type=jnp.float32)
            h = jnp.maximum(acc + b1_ref[...], 0.0).astype(jnp.bfloat16)
            s2_ref[pl.ds(row, _TM), :] = jnp.dot(
                h, w2_ref[...], preferred_element_type=jnp.float32
            ).astype(jnp.bfloat16)

        @pl.when(i >= nt)
        def _():
            acc = jnp.dot(a, s2_ref[...], preferred_element_type=jnp.float32)
            out_ref[pl.ds(row, _TM), :] = acc + b2_ref[...]

        return carry

    lax.fori_loop(0, total, _loop, 0)


def kernel(x, adj, W1, b1, W2, b2):
    n, nfeat = x.shape
    nhid = W1.shape[1]
    nout = W2.shape[1]
    w1b = W1.astype(jnp.bfloat16)
    w2b = W2.astype(jnp.bfloat16)
    b1r = b1.reshape(1, nhid)
    b2r = b2.reshape(1, nout)

    out = pl.pallas_call(
        _body,
        in_specs=[
            pl.BlockSpec(memory_space=pltpu.VMEM),
            pl.BlockSpec(memory_space=pl.ANY),
            pl.BlockSpec(memory_space=pltpu.VMEM),
            pl.BlockSpec(memory_space=pltpu.VMEM),
            pl.BlockSpec(memory_space=pltpu.VMEM),
            pl.BlockSpec(memory_space=pltpu.VMEM),
        ],
        out_specs=pl.BlockSpec(memory_space=pltpu.VMEM),
        out_shape=jax.ShapeDtypeStruct((n, nout), jnp.float32),
        scratch_shapes=[
            pltpu.VMEM((n, nhid), jnp.bfloat16),
            pltpu.VMEM((n, nout), jnp.bfloat16),
            pltpu.VMEM((2, _TM, n), jnp.float32),
            pltpu.SemaphoreType.DMA((2,)),
        ],
    )(x, adj, w1b, b1r, w2b, b2r)

    return out
